# Initial kernel scaffold; baseline (speedup 1.0000x reference)
import jax
import jax.numpy as jnp
from jax.experimental import pallas as pl


def kernel(memory, last_update, n_id):
    raise NotImplementedError("write your pallas kernel here")


# speedup vs baseline: 12.8151x; 12.8151x over previous
import functools

import jax
import jax.numpy as jnp
from jax import lax
from jax.experimental import pallas as pl
from jax.experimental.pallas import tpu as pltpu
from jax.experimental.pallas import tpu_sc as plsc

NUM_NODES = 100000
D = 128
B = 500000
NC = 2
NS = 16
NW = NC * NS
L = 16
CHUNK = 128
NFULL = B // CHUNK
TAIL = B - NFULL * CHUNK
TAIL_BASE = NFULL * CHUNK
EXTRA = NFULL % NW
BASE_CHUNKS = NFULL // NW

_mesh = plsc.VectorSubcoreMesh(core_axis_name="c", subcore_axis_name="s")


@functools.partial(
    pl.kernel,
    mesh=_mesh,
    compiler_params=pltpu.CompilerParams(needs_layout_passes=False),
    out_type=(
        jax.ShapeDtypeStruct((B, D), jnp.float32),
        jax.ShapeDtypeStruct((B,), jnp.int32),
    ),
    scratch_types=[
        pltpu.VMEM((NUM_NODES,), jnp.int32),
        pltpu.VMEM((CHUNK,), jnp.int32),
        pltpu.VMEM((CHUNK, D), jnp.float32),
        pltpu.VMEM((CHUNK,), jnp.int32),
        pltpu.VMEM((TAIL,), jnp.int32),
        pltpu.VMEM((TAIL, D), jnp.float32),
        pltpu.VMEM((TAIL,), jnp.int32),
        pltpu.SemaphoreType.DMA,
    ],
)
def _gather_kernel(mem_hbm, lu_hbm, nid_hbm, out_mem, out_lu,
                   lu_tab, idx_v, rows_v, lu_v, idx_t, rows_t, lu_t, sem0):
    wid = lax.axis_index("s") * NC + lax.axis_index("c")
    nchunks = BASE_CHUNKS + jnp.where(wid < EXTRA, 1, 0)

    pltpu.sync_copy(lu_hbm, lu_tab)

    def body(t, carry):
        base = (wid + t * NW) * CHUNK
        pltpu.sync_copy(nid_hbm.at[pl.ds(base, CHUNK)], idx_v)
        cp_rows = pltpu.async_copy(mem_hbm.at[idx_v], rows_v, sem0)
        for j in range(CHUNK // L):
            ivec = idx_v[pl.ds(j * L, L)]
            lu_v[pl.ds(j * L, L)] = plsc.load_gather(lu_tab, [ivec])
        cp_rows.wait()
        pltpu.sync_copy(rows_v, out_mem.at[pl.ds(base, CHUNK)])
        pltpu.sync_copy(lu_v, out_lu.at[pl.ds(base, CHUNK)])
        return carry

    lax.fori_loop(0, nchunks, body, 0)

    @pl.when(wid == NW - 1)
    def _tail():
        pltpu.sync_copy(nid_hbm.at[pl.ds(TAIL_BASE, TAIL)], idx_t)
        cp_rows = pltpu.async_copy(mem_hbm.at[idx_t], rows_t, sem0)
        for j in range(TAIL // L):
            ivec = idx_t[pl.ds(j * L, L)]
            lu_t[pl.ds(j * L, L)] = plsc.load_gather(lu_tab, [ivec])
        cp_rows.wait()
        pltpu.sync_copy(rows_t, out_mem.at[pl.ds(TAIL_BASE, TAIL)])
        pltpu.sync_copy(lu_t, out_lu.at[pl.ds(TAIL_BASE, TAIL)])


def kernel(memory, last_update, n_id):
    lu = last_update.astype(jnp.int32)
    nid = n_id.astype(jnp.int32)
    mem_out, lu_out = _gather_kernel(memory, lu, nid)
    return mem_out, lu_out.astype(last_update.dtype)


# speedup vs baseline: 23.8177x; 1.8586x over previous
import functools

import jax
import jax.numpy as jnp
from jax import lax
from jax.experimental import pallas as pl
from jax.experimental.pallas import tpu as pltpu
from jax.experimental.pallas import tpu_sc as plsc

NUM_NODES = 100000
D = 128
B = 500000
NC = 2
NS = 16
NW = NC * NS
CHUNK = 128
SPAN = B // NW
WSPAN = 15624
K = WSPAN // CHUNK
MINI = WSPAN - K * CHUNK
TAIL = B - NW * WSPAN
TAIL_BASE = NW * WSPAN
NBUF = 4

_mesh = plsc.VectorSubcoreMesh(core_axis_name="c", subcore_axis_name="s")


@functools.partial(
    pl.kernel,
    mesh=_mesh,
    compiler_params=pltpu.CompilerParams(needs_layout_passes=False),
    out_type=(
        jax.ShapeDtypeStruct((B, D), jnp.float32),
        jax.ShapeDtypeStruct((B,), jnp.int32),
    ),
    scratch_types=[
        pltpu.VMEM_SHARED((NUM_NODES,), jnp.int32),
        pltpu.VMEM((NBUF, CHUNK), jnp.int32),
        pltpu.VMEM((NBUF, CHUNK, D), jnp.float32),
        pltpu.VMEM((NBUF, CHUNK), jnp.int32),
        pltpu.VMEM((MINI + TAIL,), jnp.int32),
        pltpu.VMEM((MINI + TAIL, D), jnp.float32),
        pltpu.VMEM((MINI + TAIL,), jnp.int32),
        pltpu.SemaphoreType.DMA((NBUF,)),
        pltpu.SemaphoreType.DMA((NBUF,)),
        pltpu.SemaphoreType.DMA((NBUF,)),
        pltpu.SemaphoreType.DMA((NBUF,)),
        pltpu.SemaphoreType.DMA,
        pltpu.SemaphoreType.DMA,
    ],
)
def _gather_kernel(mem_hbm, lu_hbm, nid_hbm, out_mem, out_lu,
                   lu_shr, idx_v, rows_v, lu_v, idx_t, rows_t, lu_t,
                   gsem, lsem, osem, qsem, sem_a, sem_b):
    wid = lax.axis_index("s") * NC + lax.axis_index("c")
    wbase = wid * WSPAN

    @pl.when(lax.axis_index("s") == 0)
    def _stage_lu():
        pltpu.sync_copy(lu_hbm, lu_shr)

    plsc.subcore_barrier()

    def fire(k, b):
        base = wbase + k * CHUNK
        pltpu.sync_copy(nid_hbm.at[pl.ds(base, CHUNK)], idx_v.at[b])
        pltpu.async_copy(mem_hbm.at[idx_v.at[b]], rows_v.at[b], gsem.at[b])
        pltpu.async_copy(lu_shr.at[idx_v.at[b]], lu_v.at[b], lsem.at[b])

    def drain_and_write(k, b):
        base = wbase + k * CHUNK
        pltpu.make_async_copy(mem_hbm.at[idx_v.at[b]], rows_v.at[b],
                              gsem.at[b]).wait()
        pltpu.make_async_copy(lu_shr.at[idx_v.at[b]], lu_v.at[b],
                              lsem.at[b]).wait()
        pltpu.async_copy(rows_v.at[b], out_mem.at[pl.ds(base, CHUNK)],
                         osem.at[b])
        pltpu.async_copy(lu_v.at[b], out_lu.at[pl.ds(base, CHUNK)],
                         qsem.at[b])

    def wait_write(b):
        pltpu.make_async_copy(rows_v.at[b], out_mem.at[pl.ds(0, CHUNK)],
                              osem.at[b]).wait()
        pltpu.make_async_copy(lu_v.at[b], out_lu.at[pl.ds(0, CHUNK)],
                              qsem.at[b]).wait()

    fire(0, 0)
    fire(1, 1)
    fire(2, 2)
    drain_and_write(0, 0)
    fire(3, 3)
    drain_and_write(1, 1)

    def group(g, carry):
        for b in range(NBUF):
            k = g * NBUF + b
            wait_write(b)
            fire(k, b)
            pb = (b + 2) % NBUF
            drain_and_write(k - 2, pb)
        return carry

    lax.fori_loop(1, K // NBUF, group, 0)

    wait_write(0)
    fire(K - 2, 0)
    drain_and_write(K - 4, 2)
    wait_write(1)
    fire(K - 1, 1)
    drain_and_write(K - 3, 3)
    drain_and_write(K - 2, 0)
    drain_and_write(K - 1, 1)
    for b in range(NBUF):
        wait_write(b)

    mini_base = wbase + K * CHUNK

    @pl.when(wid == NW - 1)
    def _tail():
        pltpu.sync_copy(nid_hbm.at[pl.ds(mini_base, MINI + TAIL)], idx_t)
        cp_rows = pltpu.async_copy(mem_hbm.at[idx_t], rows_t, sem_a)
        cp_lu = pltpu.async_copy(lu_shr.at[idx_t], lu_t, sem_b)
        cp_rows.wait()
        cp_lu.wait()
        pltpu.sync_copy(rows_t, out_mem.at[pl.ds(mini_base, MINI + TAIL)])
        pltpu.sync_copy(lu_t, out_lu.at[pl.ds(mini_base, MINI + TAIL)])

    @pl.when(wid < NW - 1)
    def _mini():
        pltpu.sync_copy(nid_hbm.at[pl.ds(mini_base, MINI)],
                        idx_t.at[pl.ds(0, MINI)])
        cp_rows = pltpu.async_copy(mem_hbm.at[idx_t.at[pl.ds(0, MINI)]],
                                   rows_t.at[pl.ds(0, MINI)], sem_a)
        cp_lu = pltpu.async_copy(lu_shr.at[idx_t.at[pl.ds(0, MINI)]],
                                 lu_t.at[pl.ds(0, MINI)], sem_b)
        cp_rows.wait()
        cp_lu.wait()
        pltpu.sync_copy(rows_t.at[pl.ds(0, MINI)],
                        out_mem.at[pl.ds(mini_base, MINI)])
        pltpu.sync_copy(lu_t.at[pl.ds(0, MINI)],
                        out_lu.at[pl.ds(mini_base, MINI)])


def kernel(memory, last_update, n_id):
    lu = last_update.astype(jnp.int32)
    nid = n_id.astype(jnp.int32)
    mem_out, lu_out = _gather_kernel(memory, lu, nid)
    return mem_out, lu_out.astype(last_update.dtype)
